# baseline (speedup 1.0000x reference)
import functools

import jax
import jax.numpy as jnp
from jax import lax
from jax.experimental import pallas as pl
from jax.experimental.pallas import tpu as pltpu
from jax.experimental.pallas import tpu_sc as plsc

NC = 2
NS = 16
NW = NC * NS

B_TOK = 16384 * 26
K = 8
D = 16
TOK_PER_W = B_TOK // NW
CHUNK = 128
NCHUNK = TOK_PER_W // CHUNK
FIRE = 8
NBLK = NCHUNK // FIRE


def _gather_body(amp_hbm, freq_hbm, phase_hbm, idx_hbm,
                 out_a, out_w, out_p, idx_v, rows_v, sem):
    wid = lax.axis_index("s") * NC + lax.axis_index("c")
    base = wid * TOK_PER_W
    pltpu.sync_copy(idx_hbm.at[wid], idx_v)

    def gather_one(tbl, out):
        def blk(b, carry):
            descs = []
            for i in range(FIRE):
                j = b * FIRE + i
                descs.append(pltpu.async_copy(
                    tbl.at[idx_v.at[j]],
                    rows_v.at[pl.ds(j * CHUNK, CHUNK)],
                    sem))
            for dsc in descs:
                dsc.wait()
            return carry
        lax.fori_loop(0, NBLK, blk, 0, unroll=False)
        pltpu.sync_copy(rows_v, out.at[pl.ds(base, TOK_PER_W)])

    gather_one(amp_hbm, out_a)
    gather_one(freq_hbm, out_w)
    gather_one(phase_hbm, out_p)


@jax.jit
def _gather3(amplitudes, frequencies, phases, idx3d):
    row_t = jax.ShapeDtypeStruct((B_TOK, K), jnp.float32)
    mesh = plsc.VectorSubcoreMesh(core_axis_name="c", subcore_axis_name="s",
                                  num_cores=NC, num_subcores=NS)
    return pl.kernel(
        _gather_body,
        out_type=(row_t, row_t, row_t),
        mesh=mesh,
        scratch_types=(
            pltpu.VMEM((NCHUNK, CHUNK), jnp.int32),
            pltpu.VMEM((TOK_PER_W, K), jnp.float32),
            pltpu.SemaphoreType.DMA,
        ),
        compiler_params=pltpu.CompilerParams(use_tc_tiling_on_sc=False),
    )(amplitudes, frequencies, phases, idx3d)


T_BLK = 2048


def _synth_body(a_ref, w_ref, p_ref, g_ref, out_ref):
    g = g_ref[0:1, :]
    acc = jnp.zeros((T_BLK, D), jnp.float32)
    for k in range(K):
        ak = a_ref[:, k:k + 1]
        wk = w_ref[:, k:k + 1]
        pk = p_ref[:, k:k + 1]
        acc = acc + ak * jnp.sin(wk * g + pk)
    out_ref[:] = acc


@jax.jit
def _synth(a2, w2, p2, grid2d):
    nblk = B_TOK // T_BLK
    return pl.pallas_call(
        _synth_body,
        grid=(nblk,),
        in_specs=[
            pl.BlockSpec((T_BLK, K), lambda i: (i, 0)),
            pl.BlockSpec((T_BLK, K), lambda i: (i, 0)),
            pl.BlockSpec((T_BLK, K), lambda i: (i, 0)),
            pl.BlockSpec((1, D), lambda i: (0, 0)),
        ],
        out_specs=pl.BlockSpec((T_BLK, D), lambda i: (i, 0)),
        out_shape=jax.ShapeDtypeStruct((B_TOK, D), jnp.float32),
    )(a2, w2, p2, grid2d)


def kernel(x, amplitudes, frequencies, phases, grid):
    B, L = x.shape
    idx3d = x.reshape(-1).astype(jnp.int32).reshape(NW, NCHUNK, CHUNK)
    a2, w2, p2 = _gather3(amplitudes, frequencies, phases, idx3d)
    out = _synth(a2, w2, p2, grid.reshape(1, D))
    return out.reshape(B, L, D)


# speedup vs baseline: 5.5387x; 5.5387x over previous
import jax
import jax.numpy as jnp
from jax import lax
from jax.experimental import pallas as pl
from jax.experimental.pallas import tpu as pltpu
from jax.experimental.pallas import tpu_sc as plsc

NC = 2
NS = 16
NW = NC * NS

N_EMB = 1000000
B_SEQ = 16384
L_SEQ = 26
B_TOK = B_SEQ * L_SEQ
K = 8
D = 16

N_FULL_TILES = N_EMB // 128
TAIL = N_EMB - N_FULL_TILES * 128

TOK_PER_W = B_TOK // NW
CHUNK = 128
NCHUNK = TOK_PER_W // CHUNK
TCHUNK = 3328
NTCH = TOK_PER_W // TCHUNK
GPC = TCHUNK // CHUNK


def _relayout_body(a_t, w_t, p_t, out_a, out_w, out_p,
                   tile_v, out_v, tail_v, tailo_v, isem, osem):
    wid = lax.axis_index("s") * NC + lax.axis_index("c")
    lanes = lax.iota(jnp.int32, 16)
    row_idx = lanes & 7
    col0 = lanes >> 3

    def one_table(tbl, out):
        n_w = (N_FULL_TILES - wid + NW - 1) // NW

        def in_src(j, slot):
            b = wid + NW * j
            return pltpu.async_copy(tbl.at[:, pl.ds(b * 128, 128)],
                                    tile_v.at[slot], isem)

        @pl.when(n_w > 0)
        def _():
            in_src(0, 0)

        def body(j, carry):
            slot = j & 1
            pltpu.make_async_copy(tbl.at[:, pl.ds(0, 128)],
                                  tile_v.at[slot], isem).wait()

            @pl.when(j + 1 < n_w)
            def _():
                in_src(j + 1, 1 - slot)

            @pl.when(j >= 1)
            def _():
                pltpu.make_async_copy(out_v.at[slot],
                                      out.at[pl.ds(0, 1024)], osem).wait()

            for g in range(64):
                v = plsc.load_gather(tile_v.at[slot],
                                     [row_idx, col0 + (2 * g)])
                out_v[slot, pl.ds(16 * g, 16)] = v
            b = wid + NW * j
            pltpu.async_copy(out_v.at[slot],
                             out.at[pl.ds(b * 1024, 1024)], osem)
            return carry

        lax.fori_loop(0, n_w, body, 0, unroll=False)

        @pl.when(n_w >= 1)
        def _():
            pltpu.make_async_copy(out_v.at[0], out.at[pl.ds(0, 1024)],
                                  osem).wait()

        @pl.when(wid == NW - 1)
        def _():
            pltpu.sync_copy(tbl.at[:, pl.ds(N_FULL_TILES * 128, TAIL)],
                            tail_v)
            for g in range(32):
                v = plsc.load_gather(tail_v, [row_idx, col0 + (2 * g)])
                tailo_v[pl.ds(16 * g, 16)] = v
            pltpu.sync_copy(tailo_v,
                            out.at[pl.ds(N_FULL_TILES * 1024, TAIL * 8)])

    one_table(a_t, out_a)
    one_table(w_t, out_w)
    one_table(p_t, out_p)


@jax.jit
def _relayout3(a_t, w_t, p_t):
    flat_t = jax.ShapeDtypeStruct((N_EMB * K,), jnp.float32)
    mesh = plsc.VectorSubcoreMesh(core_axis_name="c", subcore_axis_name="s",
                                  num_cores=NC, num_subcores=NS)
    return pl.kernel(
        _relayout_body,
        out_type=(flat_t, flat_t, flat_t),
        mesh=mesh,
        scratch_types=(
            pltpu.VMEM((2, K, 128), jnp.float32),
            pltpu.VMEM((2, 1024), jnp.float32),
            pltpu.VMEM((K, TAIL), jnp.float32),
            pltpu.VMEM((TAIL * K,), jnp.float32),
            pltpu.SemaphoreType.DMA,
            pltpu.SemaphoreType.DMA,
        ),
        compiler_params=pltpu.CompilerParams(use_tc_tiling_on_sc=True, needs_layout_passes=False),
    )(a_t, w_t, p_t)


def _gather_body(amp_hbm, freq_hbm, phase_hbm, idx_hbm,
                 out_a, out_w, out_p, idx_v, rows_v, trans_v, sem):
    wid = lax.axis_index("s") * NC + lax.axis_index("c")
    base = wid * TOK_PER_W
    pltpu.sync_copy(idx_hbm.at[wid], idx_v)
    lanes = lax.iota(jnp.int32, 16)

    def gather_one(tbl, out):
        def chunk(t, carry):
            descs = []
            for i in range(GPC):
                j = t * GPC + i
                descs.append(pltpu.async_copy(
                    tbl.at[idx_v.at[j]],
                    rows_v.at[pl.ds(i * CHUNK, CHUNK)], sem))
            for dsc in descs:
                dsc.wait()
            for k in range(K):
                kcol = lanes * 0 + k

                def grp(g, c):
                    v = plsc.load_gather(rows_v, [lanes + 16 * g, kcol])
                    trans_v[k, pl.ds(16 * g, 16)] = v
                    return c
                lax.fori_loop(0, TCHUNK // 16, grp, 0, unroll=8)
            for k in range(K):
                pltpu.async_copy(
                    trans_v.at[k],
                    out.at[k, pl.ds(base + t * TCHUNK, TCHUNK)], sem)
            for k in range(K):
                pltpu.make_async_copy(
                    trans_v.at[0], out.at[0, pl.ds(0, TCHUNK)], sem).wait()
            return carry
        lax.fori_loop(0, NTCH, chunk, 0, unroll=False)

    gather_one(amp_hbm, out_a)
    gather_one(freq_hbm, out_w)
    gather_one(phase_hbm, out_p)


@jax.jit
def _gather3(amp_flat, freq_flat, phase_flat, idx3d):
    amp2 = amp_flat.reshape(N_EMB, K)
    freq2 = freq_flat.reshape(N_EMB, K)
    phase2 = phase_flat.reshape(N_EMB, K)
    col_t = jax.ShapeDtypeStruct((K, B_TOK), jnp.float32)
    mesh = plsc.VectorSubcoreMesh(core_axis_name="c", subcore_axis_name="s",
                                  num_cores=NC, num_subcores=NS)
    return pl.kernel(
        _gather_body,
        out_type=(col_t, col_t, col_t),
        mesh=mesh,
        scratch_types=(
            pltpu.VMEM((NCHUNK, CHUNK), jnp.int32),
            pltpu.VMEM((TCHUNK, K), jnp.float32),
            pltpu.VMEM((K, TCHUNK), jnp.float32),
            pltpu.SemaphoreType.DMA,
        ),
        compiler_params=pltpu.CompilerParams(use_tc_tiling_on_sc=False, needs_layout_passes=False),
    )(amp2, freq2, phase2, idx3d)


T_BLK = 2048


def _synth_body(a_ref, w_ref, p_ref, g_ref, out_ref):
    g = g_ref[:]
    acc = jnp.zeros((D, T_BLK), jnp.float32)
    for k in range(K):
        ak = a_ref[k:k + 1, :]
        wk = w_ref[k:k + 1, :]
        pk = p_ref[k:k + 1, :]
        acc = acc + ak * jnp.sin(wk * g + pk)
    out_ref[:] = acc[None]


@jax.jit
def _synth(a2, w2, p2, grid_col):
    nb = B_SEQ // T_BLK
    return pl.pallas_call(
        _synth_body,
        grid=(L_SEQ, nb),
        in_specs=[
            pl.BlockSpec((K, T_BLK), lambda l, b: (0, l * (B_SEQ // T_BLK) + b)),
            pl.BlockSpec((K, T_BLK), lambda l, b: (0, l * (B_SEQ // T_BLK) + b)),
            pl.BlockSpec((K, T_BLK), lambda l, b: (0, l * (B_SEQ // T_BLK) + b)),
            pl.BlockSpec((D, 1), lambda l, b: (0, 0)),
        ],
        out_specs=pl.BlockSpec((1, D, T_BLK), lambda l, b: (l, 0, b)),
        out_shape=jax.ShapeDtypeStruct((L_SEQ, D, B_SEQ), jnp.float32),
    )(a2, w2, p2, grid_col)


def kernel(x, amplitudes, frequencies, phases, grid):
    flat_a, flat_w, flat_p = _relayout3(
        amplitudes.T, frequencies.T, phases.T)
    idx3d = (x.astype(jnp.int32).T.reshape(-1)
             .reshape(NW, NCHUNK, CHUNK))
    a2, w2, p2 = _gather3(flat_a, flat_w, flat_p, idx3d)
    out3 = _synth(a2, w2, p2, grid.reshape(D, 1))
    return jnp.transpose(out3, (2, 0, 1))


# speedup vs baseline: 7.9918x; 1.4429x over previous
import jax
import jax.numpy as jnp
from jax import lax
from jax.experimental import pallas as pl
from jax.experimental.pallas import tpu as pltpu
from jax.experimental.pallas import tpu_sc as plsc

NC = 2
NS = 16
NW = NC * NS

N_EMB = 1000000
B_SEQ = 16384
L_SEQ = 26
B_TOK = B_SEQ * L_SEQ
K = 8
D = 16

N_FULL_TILES = N_EMB // 128
TAIL = N_EMB - N_FULL_TILES * 128

TOK_PER_W = B_TOK // NW
CHUNK = 128
NCHUNK = TOK_PER_W // CHUNK
TCHUNK = 3328
NTCH = TOK_PER_W // TCHUNK
GPC = TCHUNK // CHUNK


SLAB = 4
N_SLABS = N_FULL_TILES // SLAB
SLAB_W = SLAB * 128
SLAB_E = SLAB * 1024


def _relayout_body(a_t, w_t, p_t, out_a, out_w, out_p,
                   tile_v, out_v, tail_v, tailo_v, isem, osem):
    wid = lax.axis_index("s") * NC + lax.axis_index("c")
    lanes = lax.iota(jnp.int32, 16)
    row_idx = lanes & 7
    col0 = lanes >> 3

    def one_table(tbl, out):
        n_w = (N_SLABS - wid + NW - 1) // NW

        def in_src(j, slot):
            s = wid + NW * j
            return pltpu.async_copy(tbl.at[:, pl.ds(s * SLAB_W, SLAB_W)],
                                    tile_v.at[slot], isem)

        in_src(0, 0)

        def body(j, carry):
            slot = j & 1
            pltpu.make_async_copy(tbl.at[:, pl.ds(0, SLAB_W)],
                                  tile_v.at[slot], isem).wait()

            @pl.when(j + 1 < n_w)
            def _():
                in_src(j + 1, 1 - slot)

            @pl.when(j >= 1)
            def _():
                pltpu.make_async_copy(out_v.at[slot],
                                      out.at[pl.ds(0, SLAB_E)], osem).wait()

            for g in range(SLAB_E // 16):
                colbase = (g >> 6) * 128 + ((2 * g) & 127)
                v = plsc.load_gather(tile_v.at[slot],
                                     [row_idx, col0 + colbase])
                out_v[slot, pl.ds(16 * g, 16)] = v
            s = wid + NW * j
            pltpu.async_copy(out_v.at[slot],
                             out.at[pl.ds(s * SLAB_E, SLAB_E)], osem)
            return carry

        lax.fori_loop(0, n_w, body, 0, unroll=False)

        pltpu.make_async_copy(out_v.at[0], out.at[pl.ds(0, SLAB_E)],
                              osem).wait()

        @pl.when(wid == NW - 1)
        def _():
            pltpu.sync_copy(tbl.at[:, pl.ds(N_FULL_TILES * 128, TAIL)],
                            tail_v)
            for g in range(32):
                v = plsc.load_gather(tail_v, [row_idx, col0 + (2 * g)])
                tailo_v[pl.ds(16 * g, 16)] = v
            pltpu.sync_copy(tailo_v,
                            out.at[pl.ds(N_FULL_TILES * 1024, TAIL * 8)])

    one_table(a_t, out_a)
    one_table(w_t, out_w)
    one_table(p_t, out_p)


@jax.jit
def _relayout3(a_t, w_t, p_t):
    flat_t = jax.ShapeDtypeStruct((N_EMB * K,), jnp.float32)
    mesh = plsc.VectorSubcoreMesh(core_axis_name="c", subcore_axis_name="s",
                                  num_cores=NC, num_subcores=NS)
    return pl.kernel(
        _relayout_body,
        out_type=(flat_t, flat_t, flat_t),
        mesh=mesh,
        scratch_types=(
            pltpu.VMEM((2, K, SLAB_W), jnp.float32),
            pltpu.VMEM((2, SLAB_E), jnp.float32),
            pltpu.VMEM((K, TAIL), jnp.float32),
            pltpu.VMEM((TAIL * K,), jnp.float32),
            pltpu.SemaphoreType.DMA,
            pltpu.SemaphoreType.DMA,
        ),
        compiler_params=pltpu.CompilerParams(use_tc_tiling_on_sc=True, needs_layout_passes=False),
    )(a_t, w_t, p_t)


def _gather_body(amp_hbm, freq_hbm, phase_hbm, idx_hbm,
                 out_a, out_w, out_p, idx_v, rows_v, trans_v, sem):
    wid = lax.axis_index("s") * NC + lax.axis_index("c")
    base = wid * TOK_PER_W
    pltpu.sync_copy(idx_hbm.at[wid], idx_v)
    lanes = lax.iota(jnp.int32, 16)

    def gather_one(tbl, out):
        def chunk(t, carry):
            descs = []
            for i in range(GPC):
                j = t * GPC + i
                descs.append(pltpu.async_copy(
                    tbl.at[idx_v.at[j]],
                    rows_v.at[pl.ds(i * CHUNK, CHUNK)], sem))
            for dsc in descs:
                dsc.wait()
            for k in range(K):
                kcol = lanes * 0 + k

                def grp(g, c):
                    v = plsc.load_gather(rows_v, [lanes + 16 * g, kcol])
                    trans_v[k, pl.ds(16 * g, 16)] = v
                    return c
                lax.fori_loop(0, TCHUNK // 16, grp, 0, unroll=8)
            for k in range(K):
                pltpu.async_copy(
                    trans_v.at[k],
                    out.at[k, pl.ds(base + t * TCHUNK, TCHUNK)], sem)
            for k in range(K):
                pltpu.make_async_copy(
                    trans_v.at[0], out.at[0, pl.ds(0, TCHUNK)], sem).wait()
            return carry
        lax.fori_loop(0, NTCH, chunk, 0, unroll=False)

    gather_one(amp_hbm, out_a)
    gather_one(freq_hbm, out_w)
    gather_one(phase_hbm, out_p)


@jax.jit
def _gather3(amp_flat, freq_flat, phase_flat, idx3d):
    amp2 = amp_flat.reshape(N_EMB, K)
    freq2 = freq_flat.reshape(N_EMB, K)
    phase2 = phase_flat.reshape(N_EMB, K)
    col_t = jax.ShapeDtypeStruct((K, B_TOK), jnp.float32)
    mesh = plsc.VectorSubcoreMesh(core_axis_name="c", subcore_axis_name="s",
                                  num_cores=NC, num_subcores=NS)
    return pl.kernel(
        _gather_body,
        out_type=(col_t, col_t, col_t),
        mesh=mesh,
        scratch_types=(
            pltpu.VMEM((NCHUNK, CHUNK), jnp.int32),
            pltpu.VMEM((TCHUNK, K), jnp.float32),
            pltpu.VMEM((K, TCHUNK), jnp.float32),
            pltpu.SemaphoreType.DMA,
        ),
        compiler_params=pltpu.CompilerParams(use_tc_tiling_on_sc=False, needs_layout_passes=False),
    )(amp2, freq2, phase2, idx3d)


T_BLK = 2048


def _synth_body(a_ref, w_ref, p_ref, g_ref, out_ref):
    g0 = g_ref[0:1, 0:1]
    step = g_ref[1:2, 0:1] - g0
    a = a_ref[:]
    w = w_ref[:]
    ph = p_ref[:] + w * g0
    delta = w * step
    c = 2.0 * jnp.cos(delta)
    s_prev = a * jnp.sin(ph)
    s_cur = a * jnp.sin(ph + delta)
    out_ref[0, 0, :] = jnp.sum(s_prev, axis=0)
    out_ref[0, 1, :] = jnp.sum(s_cur, axis=0)
    for d in range(2, D):
        s_prev, s_cur = s_cur, c * s_cur - s_prev
        out_ref[0, d, :] = jnp.sum(s_cur, axis=0)


@jax.jit
def _synth(a2, w2, p2, grid_col):
    nb = B_SEQ // T_BLK
    return pl.pallas_call(
        _synth_body,
        grid=(L_SEQ, nb),
        in_specs=[
            pl.BlockSpec((K, T_BLK), lambda l, b: (0, l * (B_SEQ // T_BLK) + b)),
            pl.BlockSpec((K, T_BLK), lambda l, b: (0, l * (B_SEQ // T_BLK) + b)),
            pl.BlockSpec((K, T_BLK), lambda l, b: (0, l * (B_SEQ // T_BLK) + b)),
            pl.BlockSpec((D, 1), lambda l, b: (0, 0)),
        ],
        out_specs=pl.BlockSpec((1, D, T_BLK), lambda l, b: (l, 0, b)),
        out_shape=jax.ShapeDtypeStruct((L_SEQ, D, B_SEQ), jnp.float32),
    )(a2, w2, p2, grid_col)


def kernel(x, amplitudes, frequencies, phases, grid):
    flat_a, flat_w, flat_p = _relayout3(
        amplitudes.T, frequencies.T, phases.T)
    idx3d = (x.astype(jnp.int32).T.reshape(-1)
             .reshape(NW, NCHUNK, CHUNK))
    a2, w2, p2 = _gather3(flat_a, flat_w, flat_p, idx3d)
    out3 = _synth(a2, w2, p2, grid.reshape(D, 1))
    return jnp.transpose(out3, (2, 0, 1))


# speedup vs baseline: 9.8873x; 1.2372x over previous
import jax
import jax.numpy as jnp
from jax import lax
from jax.experimental import pallas as pl
from jax.experimental.pallas import tpu as pltpu
from jax.experimental.pallas import tpu_sc as plsc

NC = 2
NS = 16
NW = NC * NS

N_EMB = 1000000
B_SEQ = 16384
L_SEQ = 26
B_TOK = B_SEQ * L_SEQ
K = 8
D = 16

N_FULL_TILES = N_EMB // 128
TAIL = N_EMB - N_FULL_TILES * 128

TOK_PER_W = B_TOK // NW
CHUNK = 128
NCHUNK = TOK_PER_W // CHUNK
TCHUNK = 3328
NTCH = TOK_PER_W // TCHUNK
GPC = TCHUNK // CHUNK


SLAB = 4
N_SLABS = N_FULL_TILES // SLAB
SLAB_W = SLAB * 128
SLAB_E = SLAB * 1024


def _relayout_body(a_t, w_t, p_t, out_a, out_w, out_p,
                   tile_v, out_v, tail_v, tailo_v, isem, osem):
    wid = lax.axis_index("s") * NC + lax.axis_index("c")
    lanes = lax.iota(jnp.int32, 16)
    rowvs = [(lanes + 5 * (lanes >> 3) + rr) & 7 for rr in range(8)]
    outvs = [lanes * 8 + rowvs[rr] for rr in range(8)]

    def one_table(tbl, out):
        n_w = (N_SLABS - wid + NW - 1) // NW

        def in_src(j, slot):
            s = wid + NW * j
            return pltpu.async_copy(tbl.at[:, pl.ds(s * SLAB_W, SLAB_W)],
                                    tile_v.at[slot], isem)

        in_src(0, 0)

        def body(j, carry):
            slot = j & 1
            pltpu.make_async_copy(tbl.at[:, pl.ds(0, SLAB_W)],
                                  tile_v.at[slot], isem).wait()

            @pl.when(j + 1 < n_w)
            def _():
                in_src(j + 1, 1 - slot)

            @pl.when(j >= 1)
            def _():
                pltpu.make_async_copy(out_v.at[slot],
                                      out.at[pl.ds(0, SLAB_E)], osem).wait()

            slotv = lanes * 0 + slot
            for m in range(SLAB_W // 16):
                colv = lanes + 16 * m
                for rr in range(8):
                    v = plsc.load_gather(tile_v.at[slot], [rowvs[rr], colv])
                    plsc.store_scatter(out_v, [slotv, outvs[rr] + 128 * m], v)
            s = wid + NW * j
            pltpu.async_copy(out_v.at[slot],
                             out.at[pl.ds(s * SLAB_E, SLAB_E)], osem)
            return carry

        lax.fori_loop(0, n_w, body, 0, unroll=False)

        pltpu.make_async_copy(out_v.at[0], out.at[pl.ds(0, SLAB_E)],
                              osem).wait()

        @pl.when(wid == NW - 1)
        def _():
            pltpu.sync_copy(tbl.at[:, pl.ds(N_FULL_TILES * 128, TAIL)],
                            tail_v)
            for m in range(TAIL // 16):
                colv = lanes + 16 * m
                for rr in range(8):
                    v = plsc.load_gather(tail_v, [rowvs[rr], colv])
                    plsc.store_scatter(tailo_v, [outvs[rr] + 128 * m], v)
            pltpu.sync_copy(tailo_v,
                            out.at[pl.ds(N_FULL_TILES * 1024, TAIL * 8)])

    one_table(a_t, out_a)
    one_table(w_t, out_w)
    one_table(p_t, out_p)


@jax.jit
def _relayout3(a_t, w_t, p_t):
    flat_t = jax.ShapeDtypeStruct((N_EMB * K,), jnp.float32)
    mesh = plsc.VectorSubcoreMesh(core_axis_name="c", subcore_axis_name="s",
                                  num_cores=NC, num_subcores=NS)
    return pl.kernel(
        _relayout_body,
        out_type=(flat_t, flat_t, flat_t),
        mesh=mesh,
        scratch_types=(
            pltpu.VMEM((2, K, SLAB_W), jnp.float32),
            pltpu.VMEM((2, SLAB_E), jnp.float32),
            pltpu.VMEM((K, TAIL), jnp.float32),
            pltpu.VMEM((TAIL * K,), jnp.float32),
            pltpu.SemaphoreType.DMA,
            pltpu.SemaphoreType.DMA,
        ),
        compiler_params=pltpu.CompilerParams(use_tc_tiling_on_sc=True, needs_layout_passes=False),
    )(a_t, w_t, p_t)


def _gather_body(amp_hbm, freq_hbm, phase_hbm, idx_hbm,
                 out_a, out_w, out_p, idx_v, rows_v, trans_v, sem):
    wid = lax.axis_index("s") * NC + lax.axis_index("c")
    base = wid * TOK_PER_W
    pltpu.sync_copy(idx_hbm.at[wid], idx_v)
    lanes = lax.iota(jnp.int32, 16)
    kvs = [(lanes + 5 * (lanes >> 3) + rr) & 7 for rr in range(8)]

    def gather_one(tbl, out):
        def chunk(t, carry):
            descs = []
            for i in range(GPC):
                j = t * GPC + i
                descs.append(pltpu.async_copy(
                    tbl.at[idx_v.at[j]],
                    rows_v.at[pl.ds(i * CHUNK, CHUNK)], sem))
            for dsc in descs:
                dsc.wait()
            def grp(g, c):
                rv = lanes + 16 * g
                for rr in range(8):
                    v = plsc.load_gather(rows_v, [rv, kvs[rr]])
                    plsc.store_scatter(trans_v, [kvs[rr], rv], v)
                return c
            lax.fori_loop(0, TCHUNK // 16, grp, 0, unroll=4)
            for k in range(K):
                pltpu.async_copy(
                    trans_v.at[k],
                    out.at[k, pl.ds(base + t * TCHUNK, TCHUNK)], sem)
            for k in range(K):
                pltpu.make_async_copy(
                    trans_v.at[0], out.at[0, pl.ds(0, TCHUNK)], sem).wait()
            return carry
        lax.fori_loop(0, NTCH, chunk, 0, unroll=False)

    gather_one(amp_hbm, out_a)
    gather_one(freq_hbm, out_w)
    gather_one(phase_hbm, out_p)


@jax.jit
def _gather3(amp_flat, freq_flat, phase_flat, idx3d):
    amp2 = amp_flat.reshape(N_EMB, K)
    freq2 = freq_flat.reshape(N_EMB, K)
    phase2 = phase_flat.reshape(N_EMB, K)
    col_t = jax.ShapeDtypeStruct((K, B_TOK), jnp.float32)
    mesh = plsc.VectorSubcoreMesh(core_axis_name="c", subcore_axis_name="s",
                                  num_cores=NC, num_subcores=NS)
    return pl.kernel(
        _gather_body,
        out_type=(col_t, col_t, col_t),
        mesh=mesh,
        scratch_types=(
            pltpu.VMEM((NCHUNK, CHUNK), jnp.int32),
            pltpu.VMEM((TCHUNK, K), jnp.float32),
            pltpu.VMEM((K, TCHUNK), jnp.float32),
            pltpu.SemaphoreType.DMA,
        ),
        compiler_params=pltpu.CompilerParams(use_tc_tiling_on_sc=False, needs_layout_passes=False),
    )(amp2, freq2, phase2, idx3d)


T_BLK = 2048


def _synth_body(a_ref, w_ref, p_ref, g_ref, out_ref):
    g0 = g_ref[0:1, 0:1]
    step = g_ref[1:2, 0:1] - g0
    a = a_ref[:]
    w = w_ref[:]
    ph = p_ref[:] + w * g0
    delta = w * step
    c = 2.0 * jnp.cos(delta)
    s_prev = a * jnp.sin(ph)
    s_cur = a * jnp.sin(ph + delta)
    out_ref[0, 0, :] = jnp.sum(s_prev, axis=0)
    out_ref[0, 1, :] = jnp.sum(s_cur, axis=0)
    for d in range(2, D):
        s_prev, s_cur = s_cur, c * s_cur - s_prev
        out_ref[0, d, :] = jnp.sum(s_cur, axis=0)


@jax.jit
def _synth(a2, w2, p2, grid_col):
    nb = B_SEQ // T_BLK
    return pl.pallas_call(
        _synth_body,
        grid=(L_SEQ, nb),
        in_specs=[
            pl.BlockSpec((K, T_BLK), lambda l, b: (0, l * (B_SEQ // T_BLK) + b)),
            pl.BlockSpec((K, T_BLK), lambda l, b: (0, l * (B_SEQ // T_BLK) + b)),
            pl.BlockSpec((K, T_BLK), lambda l, b: (0, l * (B_SEQ // T_BLK) + b)),
            pl.BlockSpec((D, 1), lambda l, b: (0, 0)),
        ],
        out_specs=pl.BlockSpec((1, D, T_BLK), lambda l, b: (l, 0, b)),
        out_shape=jax.ShapeDtypeStruct((L_SEQ, D, B_SEQ), jnp.float32),
    )(a2, w2, p2, grid_col)


def kernel(x, amplitudes, frequencies, phases, grid):
    flat_a, flat_w, flat_p = _relayout3(
        amplitudes.T, frequencies.T, phases.T)
    idx3d = (x.astype(jnp.int32).T.reshape(-1)
             .reshape(NW, NCHUNK, CHUNK))
    a2, w2, p2 = _gather3(flat_a, flat_w, flat_p, idx3d)
    out3 = _synth(a2, w2, p2, grid.reshape(D, 1))
    return jnp.transpose(out3, (2, 0, 1))
